# baseline (speedup 1.0000x reference)
import functools

import jax
import jax.numpy as jnp
from jax.experimental import pallas as pl
from jax.experimental.pallas import tpu as pltpu
from jax.experimental.pallas import tpu_sc as plsc

E = 8
D = 1024
DFF = 4096
T = 4096
TILE = 128
R = 2 * T + E * TILE
NB = R // TILE
NBP = 128
W_SCAT = 32
W_GATH = 16

def _sc_mesh():
    return plsc.VectorSubcoreMesh(core_axis_name="core", subcore_axis_name="subcore")


def _gelu_exact(v):
    z = v * 0.7071067811865476
    az = jnp.abs(z)
    t = 1.0 / (1.0 + 0.3275911 * az)
    poly = t * (0.254829592 + t * (-0.284496736 + t * (1.421413741
            + t * (-1.453152027 + t * 1.061405429))))
    erf_az = 1.0 - poly * jnp.exp(-az * az)
    erf = jnp.where(z < 0.0, -erf_az, erf_az)
    return 0.5 * v * (1.0 + erf)


def _router_body(x_ref, gw_ref, r0_ref, r1_ref, g0_ref, g1_ref, bexp_ref):
    x = x_ref[...]
    gw = gw_ref[...]
    routes = jax.lax.dot_general(
        x, gw, (((1,), (1,)), ((), ())),
        preferred_element_type=jnp.float32,
        precision=jax.lax.Precision.DEFAULT)
    lane_e = jax.lax.broadcasted_iota(jnp.int32, (T, E), 1)
    m1 = jnp.max(routes, axis=1, keepdims=True)
    e0 = jnp.min(jnp.where(routes == m1, lane_e, E), axis=1, keepdims=True)
    masked = jnp.where(lane_e == e0, -jnp.inf, routes)
    m2 = jnp.max(masked, axis=1, keepdims=True)
    e1 = jnp.min(jnp.where(masked == m2, lane_e, E), axis=1, keepdims=True)
    g0 = 1.0 / (1.0 + jnp.exp(m2 - m1))
    g1 = 1.0 - g0
    oh0 = (lane_e == e0).astype(jnp.float32)
    oh1 = (lane_e == e1).astype(jnp.float32)
    cnt = oh0 + oh1
    c = cnt
    s = 1
    while s < T:
        shifted = jnp.concatenate(
            [jnp.zeros((s, E), jnp.float32), c[: T - s]], axis=0)
        c = c + shifted
        s *= 2
    ex = c - cnt
    counts = c[T - 1: T, :]
    padded = ((counts.astype(jnp.int32) + TILE - 1) // TILE) * TILE
    padf = padded.astype(jnp.float32)
    v = padf
    s = 1
    while s < E:
        v = v + jnp.concatenate(
            [jnp.zeros((1, s), jnp.float32), v[:, : E - s]], axis=1)
        s *= 2
    off = v - padf
    off_t0 = jnp.sum(oh0 * off, axis=1, keepdims=True)
    off_t1 = jnp.sum(oh1 * off, axis=1, keepdims=True)
    rank0 = jnp.sum(oh0 * ex, axis=1, keepdims=True)
    rank1 = jnp.sum(oh1 * ex, axis=1, keepdims=True)
    r0_ref[...] = (off_t0 + rank0).astype(jnp.int32)
    r1_ref[...] = (off_t1 + rank1).astype(jnp.int32)
    g0_ref[...] = g0
    g1_ref[...] = g1
    off_i = off.astype(jnp.int32)
    brow = jax.lax.broadcasted_iota(jnp.int32, (1, NBP), 1) * TILE
    acc = jnp.full((1, NBP), -1, jnp.int32)
    for e in range(E):
        acc = acc + (brow >= off_i[:, e: e + 1]).astype(jnp.int32)
    bexp_ref[...] = acc


def _router(x2d, gate_w):
    return pl.pallas_call(
        _router_body,
        out_shape=(
            jax.ShapeDtypeStruct((T, 1), jnp.int32),
            jax.ShapeDtypeStruct((T, 1), jnp.int32),
            jax.ShapeDtypeStruct((T, 1), jnp.float32),
            jax.ShapeDtypeStruct((T, 1), jnp.float32),
            jax.ShapeDtypeStruct((1, NBP), jnp.int32),
        ),
    )(x2d, gate_w)


NW = 32
TPW = T // NW
NCH_S = TPW // W_SCAT


DH = D // 2


def _dispatch(x2d, r0_3d, r1_3d):
    @functools.partial(
        pl.kernel,
        out_type=jax.ShapeDtypeStruct((R, DH), jnp.int32),
        mesh=_sc_mesh(),
        scratch_types=[
            pltpu.VMEM((NCH_S, W_SCAT), jnp.int32),
            pltpu.VMEM((NCH_S, W_SCAT), jnp.int32),
            pltpu.VMEM((W_SCAT, DH), jnp.int32),
            pltpu.SemaphoreType.DMA,
        ],
    )
    def k(x_hbm, r0_hbm, r1_hbm, xs_hbm, i0_v, i1_v, xbuf, sem):
        wid = jax.lax.axis_index("subcore") * 2 + jax.lax.axis_index("core")
        base = wid * TPW
        pltpu.sync_copy(r0_hbm.at[wid], i0_v)
        pltpu.sync_copy(r1_hbm.at[wid], i1_v)

        @pl.loop(0, NCH_S)
        def _(c):
            pltpu.sync_copy(x_hbm.at[pl.ds(base + c * W_SCAT, W_SCAT)], xbuf)
            cp0 = pltpu.async_copy(xbuf, xs_hbm.at[i0_v.at[c]], sem)
            cp1 = pltpu.async_copy(xbuf, xs_hbm.at[i1_v.at[c]], sem)
            cp0.wait()
            cp1.wait()

    return k(x2d, r0_3d, r1_3d)


def _ffn_body(bexp_ref, xs_ref, w1_ref, b1_ref, w2_ref, b2_ref, ys_ref):
    acc = jax.lax.dot_general(
        xs_ref[...], w1_ref[0], (((1,), (1,)), ((), ())),
        preferred_element_type=jnp.float32)
    h = _gelu_exact(acc + b1_ref[0]).astype(jnp.bfloat16)
    ys = jax.lax.dot_general(
        h, w2_ref[0], (((1,), (1,)), ((), ())),
        preferred_element_type=jnp.float32)
    ys_ref[...] = ys + b2_ref[0]


def _ffn(bexp, xs, w1b, b1r, w2b, b2r):
    grid_spec = pltpu.PrefetchScalarGridSpec(
        num_scalar_prefetch=1,
        grid=(NB,),
        in_specs=[
            pl.BlockSpec((TILE, D), lambda b, be: (b, 0)),
            pl.BlockSpec((1, DFF, D), lambda b, be: (be[b], 0, 0)),
            pl.BlockSpec((1, 1, DFF), lambda b, be: (be[b], 0, 0)),
            pl.BlockSpec((1, D, DFF), lambda b, be: (be[b], 0, 0)),
            pl.BlockSpec((1, 1, D), lambda b, be: (be[b], 0, 0)),
        ],
        out_specs=pl.BlockSpec((TILE, D), lambda b, be: (b, 0)),
    )
    return pl.pallas_call(
        _ffn_body,
        grid_spec=grid_spec,
        out_shape=jax.ShapeDtypeStruct((R, D), jnp.float32),
    )(bexp, xs, w1b, b1r, w2b, b2r)


RPW = 2 * T // NW
NCH_G = RPW // W_GATH


def _gather(ys, rcat_3d):
    @functools.partial(
        pl.kernel,
        out_type=jax.ShapeDtypeStruct((2 * T, D), jnp.float32),
        mesh=_sc_mesh(),
        scratch_types=[
            pltpu.VMEM((NCH_G, W_GATH), jnp.int32),
            pltpu.VMEM((W_GATH, D), jnp.float32),
            pltpu.SemaphoreType.DMA,
        ],
    )
    def k(ys_hbm, idx_hbm, ab_hbm, idx_v, buf, sem):
        wid = jax.lax.axis_index("subcore") * 2 + jax.lax.axis_index("core")
        base = wid * RPW
        pltpu.sync_copy(idx_hbm.at[wid], idx_v)

        @pl.loop(0, NCH_G)
        def _(c):
            pltpu.async_copy(ys_hbm.at[idx_v.at[c]], buf, sem).wait()
            pltpu.sync_copy(buf, ab_hbm.at[pl.ds(base + c * W_GATH, W_GATH)])

    return k(ys, rcat_3d)


_CT = 512


def _combine_body(a_ref, b_ref, g0_ref, g1_ref, o_ref):
    o_ref[...] = g0_ref[...] * a_ref[...] + g1_ref[...] * b_ref[...]


def _combine(ab, g0c, g1c):
    return pl.pallas_call(
        _combine_body,
        grid=(T // _CT,),
        in_specs=[
            pl.BlockSpec((_CT, D), lambda t: (t, 0)),
            pl.BlockSpec((_CT, D), lambda t: (t + T // _CT, 0)),
            pl.BlockSpec((_CT, 1), lambda t: (t, 0)),
            pl.BlockSpec((_CT, 1), lambda t: (t, 0)),
        ],
        out_specs=pl.BlockSpec((_CT, D), lambda t: (t, 0)),
        out_shape=jax.ShapeDtypeStruct((T, D), jnp.float32),
    )(ab, ab, g0c, g1c)


def kernel(inputs, gate_w, w1, b1, w2, b2):
    bsz, seq, _ = inputs.shape
    x2d = inputs.reshape(T, D)
    r0, r1, g0, g1, bexp_p = _router(x2d, gate_w)
    bexp = bexp_p[0, :NB]
    xi = jax.lax.bitcast_convert_type(
        x2d.astype(jnp.bfloat16).reshape(T, DH, 2), jnp.int32)
    xsi = _dispatch(xi, r0.reshape(NW, NCH_S, W_SCAT),
                    r1.reshape(NW, NCH_S, W_SCAT))
    xs = jax.lax.bitcast_convert_type(xsi, jnp.bfloat16).reshape(R, D)
    ys = _ffn(bexp, xs, w1.astype(jnp.bfloat16), b1.reshape(E, 1, DFF),
              w2.astype(jnp.bfloat16), b2.reshape(E, 1, D))
    rcat = jnp.concatenate([r0, r1], axis=0)
    ab = _gather(ys, rcat.reshape(NW, NCH_G, W_GATH))
    out = _combine(ab, g0, g1)
    return out.reshape(bsz, seq, D)


# speedup vs baseline: 1.2384x; 1.2384x over previous
import functools

import jax
import jax.numpy as jnp
from jax.experimental import pallas as pl
from jax.experimental.pallas import tpu as pltpu
from jax.experimental.pallas import tpu_sc as plsc

E = 8
D = 1024
DFF = 4096
T = 4096
TILE = 256
R = 2 * T + E * TILE
NB = R // TILE
NBP = 128
W_SCAT = 32
W_GATH = 16

def _sc_mesh():
    return plsc.VectorSubcoreMesh(core_axis_name="core", subcore_axis_name="subcore")


def _gelu_exact(v):
    z = v * 0.7071067811865476
    az = jnp.abs(z)
    t = 1.0 / (1.0 + 0.3275911 * az)
    poly = t * (0.254829592 + t * (-0.284496736 + t * (1.421413741
            + t * (-1.453152027 + t * 1.061405429))))
    erf_az = 1.0 - poly * jnp.exp(-az * az)
    erf = jnp.where(z < 0.0, -erf_az, erf_az)
    return 0.5 * v * (1.0 + erf)


def _router_body(x_ref, gw_ref, r0_ref, r1_ref, g0_ref, g1_ref, bexp_ref):
    x = x_ref[...]
    gw = gw_ref[...]
    routes = jax.lax.dot_general(
        x, gw, (((1,), (1,)), ((), ())),
        preferred_element_type=jnp.float32,
        precision=jax.lax.Precision.DEFAULT)
    lane_e = jax.lax.broadcasted_iota(jnp.int32, (T, E), 1)
    m1 = jnp.max(routes, axis=1, keepdims=True)
    e0 = jnp.min(jnp.where(routes == m1, lane_e, E), axis=1, keepdims=True)
    masked = jnp.where(lane_e == e0, -jnp.inf, routes)
    m2 = jnp.max(masked, axis=1, keepdims=True)
    e1 = jnp.min(jnp.where(masked == m2, lane_e, E), axis=1, keepdims=True)
    g0 = 1.0 / (1.0 + jnp.exp(m2 - m1))
    g1 = 1.0 - g0
    oh0 = (lane_e == e0).astype(jnp.float32)
    oh1 = (lane_e == e1).astype(jnp.float32)
    cnt = oh0 + oh1
    c = cnt
    s = 1
    while s < T:
        shifted = jnp.concatenate(
            [jnp.zeros((s, E), jnp.float32), c[: T - s]], axis=0)
        c = c + shifted
        s *= 2
    ex = c - cnt
    counts = c[T - 1: T, :]
    padded = ((counts.astype(jnp.int32) + TILE - 1) // TILE) * TILE
    padf = padded.astype(jnp.float32)
    v = padf
    s = 1
    while s < E:
        v = v + jnp.concatenate(
            [jnp.zeros((1, s), jnp.float32), v[:, : E - s]], axis=1)
        s *= 2
    off = v - padf
    off_t0 = jnp.sum(oh0 * off, axis=1, keepdims=True)
    off_t1 = jnp.sum(oh1 * off, axis=1, keepdims=True)
    rank0 = jnp.sum(oh0 * ex, axis=1, keepdims=True)
    rank1 = jnp.sum(oh1 * ex, axis=1, keepdims=True)
    r0_ref[...] = (off_t0 + rank0).astype(jnp.int32)
    r1_ref[...] = (off_t1 + rank1).astype(jnp.int32)
    g0_ref[...] = g0
    g1_ref[...] = g1
    off_i = off.astype(jnp.int32)
    brow = jax.lax.broadcasted_iota(jnp.int32, (1, NBP), 1) * TILE
    acc = jnp.full((1, NBP), -1, jnp.int32)
    for e in range(E):
        acc = acc + (brow >= off_i[:, e: e + 1]).astype(jnp.int32)
    bexp_ref[...] = acc


def _router(x2d, gate_w):
    return pl.pallas_call(
        _router_body,
        out_shape=(
            jax.ShapeDtypeStruct((T, 1), jnp.int32),
            jax.ShapeDtypeStruct((T, 1), jnp.int32),
            jax.ShapeDtypeStruct((T, 1), jnp.float32),
            jax.ShapeDtypeStruct((T, 1), jnp.float32),
            jax.ShapeDtypeStruct((1, NBP), jnp.int32),
        ),
    )(x2d, gate_w)


NW = 32
TPW = T // NW
NCH_S = TPW // W_SCAT


DH = D // 2


def _dispatch(x2d, r0_3d, r1_3d):
    @functools.partial(
        pl.kernel,
        out_type=jax.ShapeDtypeStruct((R, DH), jnp.int32),
        mesh=_sc_mesh(),
        scratch_types=[
            pltpu.VMEM((NCH_S, W_SCAT), jnp.int32),
            pltpu.VMEM((NCH_S, W_SCAT), jnp.int32),
            pltpu.VMEM((W_SCAT, DH), jnp.int32),
            pltpu.SemaphoreType.DMA,
        ],
    )
    def k(x_hbm, r0_hbm, r1_hbm, xs_hbm, i0_v, i1_v, xbuf, sem):
        wid = jax.lax.axis_index("subcore") * 2 + jax.lax.axis_index("core")
        base = wid * TPW
        pltpu.sync_copy(r0_hbm.at[wid], i0_v)
        pltpu.sync_copy(r1_hbm.at[wid], i1_v)

        @pl.loop(0, NCH_S)
        def _(c):
            pltpu.sync_copy(x_hbm.at[pl.ds(base + c * W_SCAT, W_SCAT)], xbuf)
            cp0 = pltpu.async_copy(xbuf, xs_hbm.at[i0_v.at[c]], sem)
            cp1 = pltpu.async_copy(xbuf, xs_hbm.at[i1_v.at[c]], sem)
            cp0.wait()
            cp1.wait()

    return k(x2d, r0_3d, r1_3d)


def _ffn_body(bexp_ref, xs_ref, w1_ref, b1_ref, w2_ref, b2_ref, ys_ref):
    acc = jax.lax.dot_general(
        xs_ref[...], w1_ref[0], (((1,), (1,)), ((), ())),
        preferred_element_type=jnp.float32)
    h = _gelu_exact(acc + b1_ref[0]).astype(jnp.bfloat16)
    ys = jax.lax.dot_general(
        h, w2_ref[0], (((1,), (1,)), ((), ())),
        preferred_element_type=jnp.float32)
    ys_ref[...] = ys + b2_ref[0]


def _ffn(bexp, xs, w1b, b1r, w2b, b2r):
    grid_spec = pltpu.PrefetchScalarGridSpec(
        num_scalar_prefetch=1,
        grid=(NB,),
        in_specs=[
            pl.BlockSpec((TILE, D), lambda b, be: (b, 0)),
            pl.BlockSpec((1, DFF, D), lambda b, be: (be[b], 0, 0)),
            pl.BlockSpec((1, 1, DFF), lambda b, be: (be[b], 0, 0)),
            pl.BlockSpec((1, D, DFF), lambda b, be: (be[b], 0, 0)),
            pl.BlockSpec((1, 1, D), lambda b, be: (be[b], 0, 0)),
        ],
        out_specs=pl.BlockSpec((TILE, D), lambda b, be: (b, 0)),
    )
    return pl.pallas_call(
        _ffn_body,
        grid_spec=grid_spec,
        out_shape=jax.ShapeDtypeStruct((R, D), jnp.float32),
    )(bexp, xs, w1b, b1r, w2b, b2r)


RPW = 2 * T // NW
NCH_G = RPW // W_GATH


def _gather(ys, rcat_3d):
    @functools.partial(
        pl.kernel,
        out_type=jax.ShapeDtypeStruct((2 * T, D), jnp.float32),
        mesh=_sc_mesh(),
        scratch_types=[
            pltpu.VMEM((NCH_G, W_GATH), jnp.int32),
            pltpu.VMEM((W_GATH, D), jnp.float32),
            pltpu.SemaphoreType.DMA,
        ],
    )
    def k(ys_hbm, idx_hbm, ab_hbm, idx_v, buf, sem):
        wid = jax.lax.axis_index("subcore") * 2 + jax.lax.axis_index("core")
        base = wid * RPW
        pltpu.sync_copy(idx_hbm.at[wid], idx_v)

        @pl.loop(0, NCH_G)
        def _(c):
            pltpu.async_copy(ys_hbm.at[idx_v.at[c]], buf, sem).wait()
            pltpu.sync_copy(buf, ab_hbm.at[pl.ds(base + c * W_GATH, W_GATH)])

    return k(ys, rcat_3d)


_CT = 512


def _combine_body(a_ref, b_ref, g0_ref, g1_ref, o_ref):
    o_ref[...] = g0_ref[...] * a_ref[...] + g1_ref[...] * b_ref[...]


def _combine(ab, g0c, g1c):
    return pl.pallas_call(
        _combine_body,
        grid=(T // _CT,),
        in_specs=[
            pl.BlockSpec((_CT, D), lambda t: (t, 0)),
            pl.BlockSpec((_CT, D), lambda t: (t + T // _CT, 0)),
            pl.BlockSpec((_CT, 1), lambda t: (t, 0)),
            pl.BlockSpec((_CT, 1), lambda t: (t, 0)),
        ],
        out_specs=pl.BlockSpec((_CT, D), lambda t: (t, 0)),
        out_shape=jax.ShapeDtypeStruct((T, D), jnp.float32),
    )(ab, ab, g0c, g1c)


def kernel(inputs, gate_w, w1, b1, w2, b2):
    bsz, seq, _ = inputs.shape
    x2d = inputs.reshape(T, D)
    r0, r1, g0, g1, bexp_p = _router(x2d, gate_w)
    bexp = bexp_p[0, :NB]
    xi = jax.lax.bitcast_convert_type(
        x2d.astype(jnp.bfloat16).reshape(T, DH, 2), jnp.int32)
    xsi = _dispatch(xi, r0.reshape(NW, NCH_S, W_SCAT),
                    r1.reshape(NW, NCH_S, W_SCAT))
    xs = jax.lax.bitcast_convert_type(xsi, jnp.bfloat16).reshape(R, D)
    ys = _ffn(bexp, xs, w1.astype(jnp.bfloat16), b1.reshape(E, 1, DFF),
              w2.astype(jnp.bfloat16), b2.reshape(E, 1, D))
    rcat = jnp.concatenate([r0, r1], axis=0)
    ab = _gather(ys, rcat.reshape(NW, NCH_G, W_GATH))
    out = _combine(ab, g0, g1)
    return out.reshape(bsz, seq, D)


# speedup vs baseline: 1.8114x; 1.4627x over previous
import functools

import jax
import jax.numpy as jnp
from jax.experimental import pallas as pl
from jax.experimental.pallas import tpu as pltpu
from jax.experimental.pallas import tpu_sc as plsc

E = 8
D = 1024
DFF = 4096
T = 4096
TILE = 512
R = 2 * T + E * TILE
NB = R // TILE
NBP = 128
W_SCAT = 32
W_GATH = 16

def _sc_mesh():
    return plsc.VectorSubcoreMesh(core_axis_name="core", subcore_axis_name="subcore")


def _gelu_exact(v):
    z = v * 0.7071067811865476
    az = jnp.abs(z)
    t = 1.0 / (1.0 + 0.3275911 * az)
    poly = t * (0.254829592 + t * (-0.284496736 + t * (1.421413741
            + t * (-1.453152027 + t * 1.061405429))))
    erf_az = 1.0 - poly * jnp.exp(-az * az)
    erf = jnp.where(z < 0.0, -erf_az, erf_az)
    return 0.5 * v * (1.0 + erf)


def _router_body(x_ref, gw_ref, r0_ref, r1_ref, g0_ref, g1_ref, bexp_ref):
    x = x_ref[...]
    gw = gw_ref[...]
    routes = jax.lax.dot_general(
        x, gw, (((1,), (1,)), ((), ())),
        preferred_element_type=jnp.float32,
        precision=jax.lax.Precision.DEFAULT)
    lane_e = jax.lax.broadcasted_iota(jnp.int32, (T, E), 1)
    m1 = jnp.max(routes, axis=1, keepdims=True)
    e0 = jnp.min(jnp.where(routes == m1, lane_e, E), axis=1, keepdims=True)
    masked = jnp.where(lane_e == e0, -jnp.inf, routes)
    m2 = jnp.max(masked, axis=1, keepdims=True)
    e1 = jnp.min(jnp.where(masked == m2, lane_e, E), axis=1, keepdims=True)
    g0 = 1.0 / (1.0 + jnp.exp(m2 - m1))
    g1 = 1.0 - g0
    oh0 = (lane_e == e0).astype(jnp.float32)
    oh1 = (lane_e == e1).astype(jnp.float32)
    cnt = oh0 + oh1
    c = cnt
    s = 1
    while s < T:
        shifted = jnp.concatenate(
            [jnp.zeros((s, E), jnp.float32), c[: T - s]], axis=0)
        c = c + shifted
        s *= 2
    ex = c - cnt
    counts = c[T - 1: T, :]
    padded = ((counts.astype(jnp.int32) + TILE - 1) // TILE) * TILE
    padf = padded.astype(jnp.float32)
    v = padf
    s = 1
    while s < E:
        v = v + jnp.concatenate(
            [jnp.zeros((1, s), jnp.float32), v[:, : E - s]], axis=1)
        s *= 2
    off = v - padf
    off_t0 = jnp.sum(oh0 * off, axis=1, keepdims=True)
    off_t1 = jnp.sum(oh1 * off, axis=1, keepdims=True)
    rank0 = jnp.sum(oh0 * ex, axis=1, keepdims=True)
    rank1 = jnp.sum(oh1 * ex, axis=1, keepdims=True)
    r0_ref[...] = (off_t0 + rank0).astype(jnp.int32)
    r1_ref[...] = (off_t1 + rank1).astype(jnp.int32)
    g0_ref[...] = g0
    g1_ref[...] = g1
    off_i = off.astype(jnp.int32)
    brow = jax.lax.broadcasted_iota(jnp.int32, (1, NBP), 1) * TILE
    acc = jnp.full((1, NBP), -1, jnp.int32)
    for e in range(E):
        acc = acc + (brow >= off_i[:, e: e + 1]).astype(jnp.int32)
    bexp_ref[...] = acc


def _router(x2d, gate_w):
    return pl.pallas_call(
        _router_body,
        out_shape=(
            jax.ShapeDtypeStruct((T, 1), jnp.int32),
            jax.ShapeDtypeStruct((T, 1), jnp.int32),
            jax.ShapeDtypeStruct((T, 1), jnp.float32),
            jax.ShapeDtypeStruct((T, 1), jnp.float32),
            jax.ShapeDtypeStruct((1, NBP), jnp.int32),
        ),
    )(x2d, gate_w)


NW = 32
TPW = T // NW
NCH_S = TPW // W_SCAT


def _dispatch(x2d, r0_3d, r1_3d):
    @functools.partial(
        pl.kernel,
        out_type=jax.ShapeDtypeStruct((R, D), jnp.float32),
        mesh=_sc_mesh(),
        scratch_types=[
            pltpu.VMEM((NCH_S, W_SCAT), jnp.int32),
            pltpu.VMEM((NCH_S, W_SCAT), jnp.int32),
            pltpu.VMEM((W_SCAT, D), jnp.float32),
            pltpu.SemaphoreType.DMA,
        ],
    )
    def k(x_hbm, r0_hbm, r1_hbm, xs_hbm, i0_v, i1_v, xbuf, sem):
        wid = jax.lax.axis_index("subcore") * 2 + jax.lax.axis_index("core")
        base = wid * TPW
        pltpu.sync_copy(r0_hbm.at[wid], i0_v)
        pltpu.sync_copy(r1_hbm.at[wid], i1_v)

        @pl.loop(0, NCH_S)
        def _(c):
            pltpu.sync_copy(x_hbm.at[pl.ds(base + c * W_SCAT, W_SCAT)], xbuf)
            cp0 = pltpu.async_copy(xbuf, xs_hbm.at[i0_v.at[c]], sem)
            cp1 = pltpu.async_copy(xbuf, xs_hbm.at[i1_v.at[c]], sem)
            cp0.wait()
            cp1.wait()

    return k(x2d, r0_3d, r1_3d)


def _ffn_body(bexp_ref, xs_ref, w1_ref, b1_ref, w2_ref, b2_ref, ys_ref):
    acc = jax.lax.dot_general(
        xs_ref[...].astype(jnp.bfloat16), w1_ref[0], (((1,), (1,)), ((), ())),
        preferred_element_type=jnp.float32)
    h = _gelu_exact(acc + b1_ref[0]).astype(jnp.bfloat16)
    ys = jax.lax.dot_general(
        h, w2_ref[0], (((1,), (1,)), ((), ())),
        preferred_element_type=jnp.float32)
    ys_ref[...] = ys + b2_ref[0]


def _ffn(bexp, xs, w1b, b1r, w2b, b2r):
    grid_spec = pltpu.PrefetchScalarGridSpec(
        num_scalar_prefetch=1,
        grid=(NB,),
        in_specs=[
            pl.BlockSpec((TILE, D), lambda b, be: (b, 0)),
            pl.BlockSpec((1, DFF, D), lambda b, be: (be[b], 0, 0)),
            pl.BlockSpec((1, 1, DFF), lambda b, be: (be[b], 0, 0)),
            pl.BlockSpec((1, D, DFF), lambda b, be: (be[b], 0, 0)),
            pl.BlockSpec((1, 1, D), lambda b, be: (be[b], 0, 0)),
        ],
        out_specs=pl.BlockSpec((TILE, D), lambda b, be: (b, 0)),
    )
    return pl.pallas_call(
        _ffn_body,
        grid_spec=grid_spec,
        out_shape=jax.ShapeDtypeStruct((R, D), jnp.float32),
    )(bexp, xs, w1b, b1r, w2b, b2r)


RPW = 2 * T // NW
NCH_G = RPW // W_GATH


def _gather(ys, rcat_3d):
    @functools.partial(
        pl.kernel,
        out_type=jax.ShapeDtypeStruct((2 * T, D), jnp.float32),
        mesh=_sc_mesh(),
        scratch_types=[
            pltpu.VMEM((NCH_G, W_GATH), jnp.int32),
            pltpu.VMEM((W_GATH, D), jnp.float32),
            pltpu.SemaphoreType.DMA,
        ],
    )
    def k(ys_hbm, idx_hbm, ab_hbm, idx_v, buf, sem):
        wid = jax.lax.axis_index("subcore") * 2 + jax.lax.axis_index("core")
        base = wid * RPW
        pltpu.sync_copy(idx_hbm.at[wid], idx_v)

        @pl.loop(0, NCH_G)
        def _(c):
            pltpu.async_copy(ys_hbm.at[idx_v.at[c]], buf, sem).wait()
            pltpu.sync_copy(buf, ab_hbm.at[pl.ds(base + c * W_GATH, W_GATH)])

    return k(ys, rcat_3d)


_CT = 512


def _combine_body(a_ref, b_ref, g0_ref, g1_ref, o_ref):
    o_ref[...] = g0_ref[...] * a_ref[...] + g1_ref[...] * b_ref[...]


def _combine(ab, g0c, g1c):
    return pl.pallas_call(
        _combine_body,
        grid=(T // _CT,),
        in_specs=[
            pl.BlockSpec((_CT, D), lambda t: (t, 0)),
            pl.BlockSpec((_CT, D), lambda t: (t + T // _CT, 0)),
            pl.BlockSpec((_CT, 1), lambda t: (t, 0)),
            pl.BlockSpec((_CT, 1), lambda t: (t, 0)),
        ],
        out_specs=pl.BlockSpec((_CT, D), lambda t: (t, 0)),
        out_shape=jax.ShapeDtypeStruct((T, D), jnp.float32),
    )(ab, ab, g0c, g1c)


def kernel(inputs, gate_w, w1, b1, w2, b2):
    bsz, seq, _ = inputs.shape
    x2d = inputs.reshape(T, D)
    r0, r1, g0, g1, bexp_p = _router(x2d, gate_w)
    bexp = bexp_p[0, :NB]
    xs = _dispatch(x2d, r0.reshape(NW, NCH_S, W_SCAT),
                   r1.reshape(NW, NCH_S, W_SCAT))
    ys = _ffn(bexp, xs, w1.astype(jnp.bfloat16), b1.reshape(E, 1, DFF),
              w2.astype(jnp.bfloat16), b2.reshape(E, 1, D))
    rcat = jnp.concatenate([r0, r1], axis=0)
    ab = _gather(ys, rcat.reshape(NW, NCH_G, W_GATH))
    out = _combine(ab, g0, g1)
    return out.reshape(bsz, seq, D)


# speedup vs baseline: 1.8398x; 1.0157x over previous
import functools

import jax
import jax.numpy as jnp
from jax.experimental import pallas as pl
from jax.experimental.pallas import tpu as pltpu
from jax.experimental.pallas import tpu_sc as plsc

E = 8
D = 1024
DFF = 4096
T = 4096
TILE = 512
R = 2 * T + E * TILE
NB = R // TILE
NBP = 128
W_SCAT = 64
W_GATH = 32

def _sc_mesh():
    return plsc.VectorSubcoreMesh(core_axis_name="core", subcore_axis_name="subcore")


def _gelu_exact(v):
    z = v * 0.7071067811865476
    az = jnp.abs(z)
    t = 1.0 / (1.0 + 0.3275911 * az)
    poly = t * (0.254829592 + t * (-0.284496736 + t * (1.421413741
            + t * (-1.453152027 + t * 1.061405429))))
    erf_az = 1.0 - poly * jnp.exp(-az * az)
    erf = jnp.where(z < 0.0, -erf_az, erf_az)
    return 0.5 * v * (1.0 + erf)


def _router_body(x_ref, gw_ref, r0_ref, r1_ref, g0_ref, g1_ref, bexp_ref):
    x = x_ref[...]
    gw = gw_ref[...]
    routes = jax.lax.dot_general(
        x, gw, (((1,), (1,)), ((), ())),
        preferred_element_type=jnp.float32,
        precision=jax.lax.Precision.DEFAULT)
    lane_e = jax.lax.broadcasted_iota(jnp.int32, (T, E), 1)
    m1 = jnp.max(routes, axis=1, keepdims=True)
    e0 = jnp.min(jnp.where(routes == m1, lane_e, E), axis=1, keepdims=True)
    masked = jnp.where(lane_e == e0, -jnp.inf, routes)
    m2 = jnp.max(masked, axis=1, keepdims=True)
    e1 = jnp.min(jnp.where(masked == m2, lane_e, E), axis=1, keepdims=True)
    g0 = 1.0 / (1.0 + jnp.exp(m2 - m1))
    g1 = 1.0 - g0
    oh0 = (lane_e == e0).astype(jnp.float32)
    oh1 = (lane_e == e1).astype(jnp.float32)
    cnt = oh0 + oh1
    c = cnt
    s = 1
    while s < T:
        shifted = jnp.concatenate(
            [jnp.zeros((s, E), jnp.float32), c[: T - s]], axis=0)
        c = c + shifted
        s *= 2
    ex = c - cnt
    counts = c[T - 1: T, :]
    padded = ((counts.astype(jnp.int32) + TILE - 1) // TILE) * TILE
    padf = padded.astype(jnp.float32)
    v = padf
    s = 1
    while s < E:
        v = v + jnp.concatenate(
            [jnp.zeros((1, s), jnp.float32), v[:, : E - s]], axis=1)
        s *= 2
    off = v - padf
    off_t0 = jnp.sum(oh0 * off, axis=1, keepdims=True)
    off_t1 = jnp.sum(oh1 * off, axis=1, keepdims=True)
    rank0 = jnp.sum(oh0 * ex, axis=1, keepdims=True)
    rank1 = jnp.sum(oh1 * ex, axis=1, keepdims=True)
    r0_ref[...] = (off_t0 + rank0).astype(jnp.int32)
    r1_ref[...] = (off_t1 + rank1).astype(jnp.int32)
    g0_ref[...] = g0
    g1_ref[...] = g1
    off_i = off.astype(jnp.int32)
    brow = jax.lax.broadcasted_iota(jnp.int32, (1, NBP), 1) * TILE
    acc = jnp.full((1, NBP), -1, jnp.int32)
    for e in range(E):
        acc = acc + (brow >= off_i[:, e: e + 1]).astype(jnp.int32)
    bexp_ref[...] = acc


def _router(x2d, gate_w):
    return pl.pallas_call(
        _router_body,
        out_shape=(
            jax.ShapeDtypeStruct((T, 1), jnp.int32),
            jax.ShapeDtypeStruct((T, 1), jnp.int32),
            jax.ShapeDtypeStruct((T, 1), jnp.float32),
            jax.ShapeDtypeStruct((T, 1), jnp.float32),
            jax.ShapeDtypeStruct((1, NBP), jnp.int32),
        ),
    )(x2d, gate_w)


NW = 32
TPW = T // NW
NCH_S = TPW // W_SCAT


def _dispatch(x2d, r0_3d, r1_3d):
    @functools.partial(
        pl.kernel,
        out_type=jax.ShapeDtypeStruct((R, D), jnp.float32),
        mesh=_sc_mesh(),
        scratch_types=[
            pltpu.VMEM((NCH_S, W_SCAT), jnp.int32),
            pltpu.VMEM((NCH_S, W_SCAT), jnp.int32),
            pltpu.VMEM((W_SCAT, D), jnp.float32),
            pltpu.SemaphoreType.DMA,
        ],
    )
    def k(x_hbm, r0_hbm, r1_hbm, xs_hbm, i0_v, i1_v, xbuf, sem):
        wid = jax.lax.axis_index("subcore") * 2 + jax.lax.axis_index("core")
        base = wid * TPW
        pltpu.sync_copy(r0_hbm.at[wid], i0_v)
        pltpu.sync_copy(r1_hbm.at[wid], i1_v)

        @pl.loop(0, NCH_S)
        def _(c):
            pltpu.sync_copy(x_hbm.at[pl.ds(base + c * W_SCAT, W_SCAT)], xbuf)
            cp0 = pltpu.async_copy(xbuf, xs_hbm.at[i0_v.at[c]], sem)
            cp1 = pltpu.async_copy(xbuf, xs_hbm.at[i1_v.at[c]], sem)
            cp0.wait()
            cp1.wait()

    return k(x2d, r0_3d, r1_3d)


def _ffn_body(bexp_ref, xs_ref, w1_ref, b1_ref, w2_ref, b2_ref, ys_ref):
    acc = jax.lax.dot_general(
        xs_ref[...].astype(jnp.bfloat16), w1_ref[0], (((1,), (1,)), ((), ())),
        preferred_element_type=jnp.float32)
    h = _gelu_exact(acc + b1_ref[0]).astype(jnp.bfloat16)
    ys = jax.lax.dot_general(
        h, w2_ref[0], (((1,), (1,)), ((), ())),
        preferred_element_type=jnp.float32)
    ys_ref[...] = ys + b2_ref[0]


def _ffn(bexp, xs, w1b, b1r, w2b, b2r):
    grid_spec = pltpu.PrefetchScalarGridSpec(
        num_scalar_prefetch=1,
        grid=(NB,),
        in_specs=[
            pl.BlockSpec((TILE, D), lambda b, be: (b, 0)),
            pl.BlockSpec((1, DFF, D), lambda b, be: (be[b], 0, 0)),
            pl.BlockSpec((1, 1, DFF), lambda b, be: (be[b], 0, 0)),
            pl.BlockSpec((1, D, DFF), lambda b, be: (be[b], 0, 0)),
            pl.BlockSpec((1, 1, D), lambda b, be: (be[b], 0, 0)),
        ],
        out_specs=pl.BlockSpec((TILE, D), lambda b, be: (b, 0)),
    )
    return pl.pallas_call(
        _ffn_body,
        grid_spec=grid_spec,
        out_shape=jax.ShapeDtypeStruct((R, D), jnp.float32),
    )(bexp, xs, w1b, b1r, w2b, b2r)


RPW = 2 * T // NW
NCH_G = RPW // W_GATH


def _gather(ys, rcat_3d):
    @functools.partial(
        pl.kernel,
        out_type=jax.ShapeDtypeStruct((2 * T, D), jnp.float32),
        mesh=_sc_mesh(),
        scratch_types=[
            pltpu.VMEM((NCH_G, W_GATH), jnp.int32),
            pltpu.VMEM((W_GATH, D), jnp.float32),
            pltpu.SemaphoreType.DMA,
        ],
    )
    def k(ys_hbm, idx_hbm, ab_hbm, idx_v, buf, sem):
        wid = jax.lax.axis_index("subcore") * 2 + jax.lax.axis_index("core")
        base = wid * RPW
        pltpu.sync_copy(idx_hbm.at[wid], idx_v)

        @pl.loop(0, NCH_G)
        def _(c):
            pltpu.async_copy(ys_hbm.at[idx_v.at[c]], buf, sem).wait()
            pltpu.sync_copy(buf, ab_hbm.at[pl.ds(base + c * W_GATH, W_GATH)])

    return k(ys, rcat_3d)


_CT = 512


def _combine_body(a_ref, b_ref, g0_ref, g1_ref, o_ref):
    o_ref[...] = g0_ref[...] * a_ref[...] + g1_ref[...] * b_ref[...]


def _combine(ab, g0c, g1c):
    return pl.pallas_call(
        _combine_body,
        grid=(T // _CT,),
        in_specs=[
            pl.BlockSpec((_CT, D), lambda t: (t, 0)),
            pl.BlockSpec((_CT, D), lambda t: (t + T // _CT, 0)),
            pl.BlockSpec((_CT, 1), lambda t: (t, 0)),
            pl.BlockSpec((_CT, 1), lambda t: (t, 0)),
        ],
        out_specs=pl.BlockSpec((_CT, D), lambda t: (t, 0)),
        out_shape=jax.ShapeDtypeStruct((T, D), jnp.float32),
    )(ab, ab, g0c, g1c)


def kernel(inputs, gate_w, w1, b1, w2, b2):
    bsz, seq, _ = inputs.shape
    x2d = inputs.reshape(T, D)
    r0, r1, g0, g1, bexp_p = _router(x2d, gate_w)
    bexp = bexp_p[0, :NB]
    xs = _dispatch(x2d, r0.reshape(NW, NCH_S, W_SCAT),
                   r1.reshape(NW, NCH_S, W_SCAT))
    ys = _ffn(bexp, xs, w1.astype(jnp.bfloat16), b1.reshape(E, 1, DFF),
              w2.astype(jnp.bfloat16), b2.reshape(E, 1, D))
    rcat = jnp.concatenate([r0, r1], axis=0)
    ab = _gather(ys, rcat.reshape(NW, NCH_G, W_GATH))
    out = _combine(ab, g0, g1)
    return out.reshape(bsz, seq, D)
